# baseline (speedup 1.0000x reference)
import dataclasses
import functools

import jax
import jax.numpy as jnp
from jax import lax
from jax.experimental import pallas as pl
from jax.experimental.pallas import tpu as pltpu
from jax.experimental.pallas import tpu_sc as plsc

_NC = 2
_NS = 16
_NW = _NC * _NS
_LANES = 16
_CHUNK1 = 256
_SUB1 = _CHUNK1 // 128
_CHUNK2 = 128


def _sc_compiler_params():
    cp = pltpu.CompilerParams()
    fields = pltpu.CompilerParams.__dataclass_fields__
    if "needs_layout_passes" in fields:
        cp = dataclasses.replace(cp, needs_layout_passes=False)
    if "use_tc_tiling_on_sc" in fields:
        cp = dataclasses.replace(cp, use_tc_tiling_on_sc=False)
    return cp


def _dot(a, b):
    return jnp.dot(a, b, precision=jax.lax.Precision.HIGHEST,
                   preferred_element_type=jnp.float32)


def _elu(v):
    return jnp.where(v > 0, v, jnp.exp(jnp.minimum(v, 0.0)) - 1.0)


def _lrelu(v):
    return jnp.maximum(v, 0.2 * v)


def _cmax_update(i, cacc_ref, bmv):
    @pl.when(i == 0)
    def _():
        cacc_ref[...] = bmv

    @pl.when(i > 0)
    def _():
        cacc_ref[...] = jnp.maximum(cacc_ref[...], bmv)


def _prep1_body(x_ref, w1_ref, as1_ref, ad1_ref, t1_ref, c_ref, cacc_ref):
    i = pl.program_id(0)
    x = x_ref[...]
    w1 = w1_ref[...]
    vs = jnp.stack([_dot(w1[:, :32], as1_ref[0, :]),
                    _dot(w1[:, 32:], as1_ref[1, :])], axis=1)
    vd = jnp.stack([_dot(w1[:, :32], ad1_ref[0, :]),
                    _dot(w1[:, 32:], ad1_ref[1, :])], axis=1)
    es = _dot(x, vs)
    ed = _dot(x, vd)
    rblk = x.shape[0]
    one = jnp.ones((rblk, 1), jnp.float32)
    zero = jnp.zeros((rblk, 1), jnp.float32)
    t1_ref[...] = jnp.concatenate([x, one, es, ed, zero], axis=1)
    bm = jnp.stack([jnp.max(es[:, 0]), jnp.max(es[:, 1]),
                    jnp.max(ed[:, 0]), jnp.max(ed[:, 1])])
    bm8 = jnp.concatenate([bm, jnp.zeros((4,), jnp.float32)])
    _cmax_update(i, cacc_ref, jnp.broadcast_to(bm8[:, None], (8, 128)))

    @pl.when(i == pl.num_programs(0) - 1)
    def _():
        r0 = _lrelu(cacc_ref[0, :] + cacc_ref[2, :])
        r1 = _lrelu(cacc_ref[1, :] + cacc_ref[3, :])
        z = jnp.zeros((128,), jnp.float32)
        c_ref[...] = jnp.stack([r0, r1, z, z, z, z, z, z])


def _prep2_body(n, rblk, acc_ref, w1_ref, b1_ref, g1_ref, be1_ref, w2_ref,
                as2_ref, ad2_ref, t2a_ref, t2b_ref, c_ref, cacc_ref):
    s = acc_ref[0] + acc_ref[1]
    m1 = s[:, 0:10]
    d1 = s[:, 10:11]
    m2 = s[:, 16:26]
    d2 = s[:, 26:27]
    w1 = w1_ref[...]
    o1 = _dot(m1, w1[:, :32])
    o2 = _dot(m2, w1[:, 32:])
    o1 = jnp.where(d1 > 0, o1 / d1, 0.0)
    o2 = jnp.where(d2 > 0, o2 / d2, 0.0)
    o = jnp.concatenate([o1, o2], axis=1) + b1_ref[...][None, :]
    z = o * (g1_ref[...][None, :] / jnp.sqrt(1.0 + 1e-5)) + be1_ref[...][None, :]
    z = _elu(z)
    h2 = _dot(z, w2_ref[...])
    es2 = _dot(h2, as2_ref[0, :])
    ed2 = _dot(h2, ad2_ref[0, :])
    i = pl.program_id(0)
    row = lax.broadcasted_iota(jnp.int32, (rblk, 1), 0) + i * rblk
    mask = row < n
    h2 = jnp.where(mask, h2, 0.0)
    es2 = jnp.where(mask[:, 0], es2, 0.0)
    ed2 = jnp.where(mask[:, 0], ed2, 0.0)
    zc = jnp.zeros((rblk, 1), jnp.float32)
    one = jnp.where(mask, 1.0, 0.0)
    t2a_ref[...] = jnp.concatenate(
        [h2, es2[:, None], jnp.zeros((rblk, 15), jnp.float32)], axis=1)
    t2b_ref[...] = jnp.concatenate(
        [zc, ed2[:, None], one, jnp.zeros((rblk, 13), jnp.float32)], axis=1)
    bm = jnp.stack([jnp.max(es2), jnp.max(ed2)])
    bm8 = jnp.concatenate([bm, jnp.zeros((6,), jnp.float32)])
    _cmax_update(i, cacc_ref, jnp.broadcast_to(bm8[:, None], (8, 128)))

    @pl.when(i == pl.num_programs(0) - 1)
    def _():
        r0 = _lrelu(cacc_ref[0, :] + cacc_ref[1, :])
        z = jnp.zeros((128,), jnp.float32)
        c_ref[...] = jnp.stack([r0, z, z, z, z, z, z, z])


def _final_body(acc_ref, b2_ref, wc_ref, bc_ref, y_ref):
    s = acc_ref[0] + acc_ref[1]
    m = s[:, 0:16]
    d = s[:, 18:19]
    o = jnp.where(d > 0, m / d, 0.0) + b2_ref[...][None, :]
    o = _elu(o)
    y_ref[...] = _dot(o, wc_ref[...]) + bc_ref[...][None, :]


def _sc_pass1(t1, idx4, carr, zeros, npad, steps):
    rpt = npad // _NS
    mesh = plsc.VectorSubcoreMesh(core_axis_name="c", subcore_axis_name="s")

    @functools.partial(
        pl.kernel,
        out_type=jax.ShapeDtypeStruct((_NC, npad, 32), jnp.float32),
        mesh=mesh,
        compiler_params=_sc_compiler_params(),
        scratch_types=[
            pltpu.VMEM((8, 128), jnp.int32),
            pltpu.VMEM((_CHUNK1, 16), jnp.float32),
            pltpu.VMEM((_CHUNK1, 16), jnp.float32),
            pltpu.VMEM((_CHUNK1, 16), jnp.float32),
            pltpu.VMEM((_CHUNK1, 16), jnp.float32),
            pltpu.VMEM((_CHUNK1, 32), jnp.float32),
            pltpu.VMEM((2, 16), jnp.float32),
            pltpu.VMEM_SHARED((npad, 32), jnp.float32),
            pltpu.SemaphoreType.DMA,
            pltpu.SemaphoreType.DMA,
        ],
    )
    def kern(t1_hbm, idx_hbm, c_hbm, z_hbm, out_hbm,
             sidi, srows0, srows1, drows0, drows1, msg,
             cbuf, acc, g0, g1):
        cid = lax.axis_index("c")
        sid = lax.axis_index("s")
        w = cid * _NS + sid
        pltpu.sync_copy(z_hbm.at[pl.ds(sid * rpt, rpt)],
                        acc.at[pl.ds(sid * rpt, rpt)])
        pltpu.sync_copy(c_hbm, cbuf)
        plsc.subcore_barrier()
        iota = lax.iota(jnp.int32, _LANES)
        c1v = cbuf[0, :]
        c2v = cbuf[1, :]
        col11 = jnp.full((_LANES,), 11, jnp.int32)
        col12 = jnp.full((_LANES,), 12, jnp.int32)
        col13 = jnp.full((_LANES,), 13, jnp.int32)
        col14 = jnp.full((_LANES,), 14, jnp.int32)
        bufs = [(0, srows0, drows0, g0),
                (4, srows1, drows1, g1)]

        def prefetch(b):
            off, srows, drows, sem = bufs[b]
            cps = []
            for j in range(_SUB1):
                cps.append(pltpu.async_copy(t1_hbm.at[sidi.at[off + j]],
                                            srows.at[pl.ds(j * 128, 128)],
                                            sem))
                cps.append(pltpu.async_copy(t1_hbm.at[sidi.at[off + 2 + j]],
                                            drows.at[pl.ds(j * 128, 128)],
                                            sem))
            return cps

        def process(b, cps):
            off, srows, drows, _ = bufs[b]
            for cp in cps:
                cp.wait()

            @pl.loop(0, _CHUNK1 // _LANES)
            def _(g):
                r0 = g * _LANES
                rows = r0 + iota
                es1 = plsc.load_gather(srows, [rows, col11])
                es2 = plsc.load_gather(srows, [rows, col12])
                ed1 = plsc.load_gather(drows, [rows, col13])
                ed2 = plsc.load_gather(drows, [rows, col14])
                e1 = es1 + ed1
                e1 = jnp.maximum(e1, 0.2 * e1)
                e2 = es2 + ed2
                e2 = jnp.maximum(e2, 0.2 * e2)
                p1 = jnp.exp(e1 - c1v)
                p2 = jnp.exp(e2 - c2v)
                for i in range(_LANES):
                    srow = srows[r0 + i, :]
                    msg[r0 + i, 0:16] = srow * p1[i]
                    msg[r0 + i, 16:32] = srow * p2[i]

            for j in range(_SUB1):
                pltpu.sync_copy(msg.at[pl.ds(j * 128, 128)],
                                acc.at[sidi.at[off + 2 + j]], add=True)

        @pl.loop(0, steps // 2)
        def _(k):
            base = (w * steps + 2 * k) * 4
            pltpu.sync_copy(idx_hbm.at[pl.ds(base, 8)], sidi)
            h0 = prefetch(0)
            h1 = prefetch(1)
            process(0, h0)
            process(1, h1)

        plsc.subcore_barrier()
        pltpu.sync_copy(acc.at[pl.ds(sid * rpt, rpt)],
                        out_hbm.at[cid, pl.ds(sid * rpt, rpt)])

    return kern(t1, idx4, carr, zeros)


def _sc_pass2(t2a, t2b, idx2, carr, zeros, npad, steps2):
    rpt = npad // _NS
    mesh = plsc.VectorSubcoreMesh(core_axis_name="c", subcore_axis_name="s")

    @functools.partial(
        pl.kernel,
        out_type=jax.ShapeDtypeStruct((_NC, npad, 32), jnp.float32),
        mesh=mesh,
        compiler_params=_sc_compiler_params(),
        scratch_types=[
            pltpu.VMEM((4, 128), jnp.int32),
            pltpu.VMEM((_CHUNK2, 32), jnp.float32),
            pltpu.VMEM((_CHUNK2, 32), jnp.float32),
            pltpu.VMEM((_CHUNK2, 16), jnp.float32),
            pltpu.VMEM((_CHUNK2, 16), jnp.float32),
            pltpu.VMEM((_CHUNK2, 32), jnp.float32),
            pltpu.VMEM((2, 16), jnp.float32),
            pltpu.VMEM_SHARED((npad, 32), jnp.float32),
            pltpu.SemaphoreType.DMA,
            pltpu.SemaphoreType.DMA,
        ],
    )
    def kern(t2a_hbm, t2b_hbm, idx_hbm, c_hbm, z_hbm, out_hbm,
             sidi, ar0, ar1, cr0, cr1, msg,
             cbuf, acc, g0, g1):
        cid = lax.axis_index("c")
        sid = lax.axis_index("s")
        w = cid * _NS + sid
        pltpu.sync_copy(z_hbm.at[pl.ds(sid * rpt, rpt)],
                        acc.at[pl.ds(sid * rpt, rpt)])
        pltpu.sync_copy(c_hbm, cbuf)
        plsc.subcore_barrier()
        iota = lax.iota(jnp.int32, _LANES)
        cv = cbuf[0, :]
        col16 = jnp.full((_LANES,), 16, jnp.int32)
        col1 = jnp.full((_LANES,), 1, jnp.int32)
        bufs = [(0, ar0, cr0, g0),
                (2, ar1, cr1, g1)]

        def prefetch(b):
            off, ar, cr, sem = bufs[b]
            return [
                pltpu.async_copy(t2a_hbm.at[sidi.at[off]], ar, sem),
                pltpu.async_copy(t2b_hbm.at[sidi.at[off + 1]], cr, sem),
            ]

        def process(b, cps):
            off, ar, cr, _ = bufs[b]
            for cp in cps:
                cp.wait()

            @pl.loop(0, _CHUNK2 // _LANES)
            def _(g):
                r0 = g * _LANES
                rows = r0 + iota
                es = plsc.load_gather(ar, [rows, col16])
                ed = plsc.load_gather(cr, [rows, col1])
                e = es + ed
                e = jnp.maximum(e, 0.2 * e)
                pv = jnp.exp(e - cv)
                for i in range(_LANES):
                    p = pv[i]
                    msg[r0 + i, 0:16] = ar[r0 + i, 0:16] * p
                    msg[r0 + i, 16:32] = cr[r0 + i, :] * p

            pltpu.sync_copy(msg, acc.at[sidi.at[off + 1]], add=True)

        @pl.loop(0, steps2 // 2)
        def _(k):
            base = (w * steps2 + 2 * k) * 2
            pltpu.sync_copy(idx_hbm.at[pl.ds(base, 4)], sidi)
            h0 = prefetch(0)
            h1 = prefetch(1)
            process(0, h0)
            process(1, h1)

        plsc.subcore_barrier()
        pltpu.sync_copy(acc.at[pl.ds(sid * rpt, rpt)],
                        out_hbm.at[cid, pl.ds(sid * rpt, rpt)])

    return kern(t2a, t2b, idx2, carr, zeros)


def kernel(x, edge_index, W1, a_src1, a_dst1, b1, g1, be1,
           W2, a_src2, a_dst2, b2, Wc, bc):
    n = x.shape[0]
    e = edge_index.shape[1]
    npad = ((n + 1 + 127) // 128) * 128
    two = 2 * _CHUNK1
    ew = ((e + _NW * two - 1) // (_NW * two)) * two
    steps = ew // _CHUNK1
    steps2 = ew // _CHUNK2
    ep = ew * _NW

    src = edge_index[0]
    dst = edge_index[1]
    pad = jnp.full((ep - e,), n, jnp.int32)
    srcp = jnp.concatenate([src, pad])
    dstp = jnp.concatenate([dst, pad])
    srcr = srcp.reshape(_NW * steps, _SUB1, 128)
    dstr = dstp.reshape(_NW * steps, _SUB1, 128)
    idx4 = jnp.concatenate([srcr, dstr], axis=1).reshape(_NW * steps * 4, 128)
    srcr2 = srcp.reshape(_NW * steps2, 1, 128)
    dstr2 = dstp.reshape(_NW * steps2, 1, 128)
    idx2 = jnp.concatenate([srcr2, dstr2], axis=1).reshape(_NW * steps2 * 2,
                                                           128)
    xp = jnp.pad(x, ((0, npad - n), (0, 0)))
    zeros = jnp.zeros((npad, 32), jnp.float32)

    rblk = npad // 8
    t1, c1out = pl.pallas_call(
        _prep1_body,
        grid=(npad // rblk,),
        in_specs=[
            pl.BlockSpec((rblk, 10), lambda i: (i, 0)),
            pl.BlockSpec((10, 64), lambda i: (0, 0)),
            pl.BlockSpec((2, 32), lambda i: (0, 0)),
            pl.BlockSpec((2, 32), lambda i: (0, 0)),
        ],
        out_specs=[
            pl.BlockSpec((rblk, 16), lambda i: (i, 0)),
            pl.BlockSpec((8, 128), lambda i: (0, 0)),
        ],
        out_shape=[
            jax.ShapeDtypeStruct((npad, 16), jnp.float32),
            jax.ShapeDtypeStruct((8, 128), jnp.float32),
        ],
        scratch_shapes=[pltpu.VMEM((8, 128), jnp.float32)],
    )(xp, W1, a_src1, a_dst1)
    carr1 = c1out[:2, :16]

    acc1 = _sc_pass1(t1, idx4, carr1, zeros, npad, steps)

    t2a, t2b, c2out = pl.pallas_call(
        functools.partial(_prep2_body, n, rblk),
        grid=(npad // rblk,),
        in_specs=[
            pl.BlockSpec((2, rblk, 32), lambda i: (0, i, 0)),
            pl.BlockSpec((10, 64), lambda i: (0, 0)),
            pl.BlockSpec((64,), lambda i: (0,)),
            pl.BlockSpec((64,), lambda i: (0,)),
            pl.BlockSpec((64,), lambda i: (0,)),
            pl.BlockSpec((64, 16), lambda i: (0, 0)),
            pl.BlockSpec((1, 16), lambda i: (0, 0)),
            pl.BlockSpec((1, 16), lambda i: (0, 0)),
        ],
        out_specs=[
            pl.BlockSpec((rblk, 32), lambda i: (i, 0)),
            pl.BlockSpec((rblk, 16), lambda i: (i, 0)),
            pl.BlockSpec((8, 128), lambda i: (0, 0)),
        ],
        out_shape=[
            jax.ShapeDtypeStruct((npad, 32), jnp.float32),
            jax.ShapeDtypeStruct((npad, 16), jnp.float32),
            jax.ShapeDtypeStruct((8, 128), jnp.float32),
        ],
        scratch_shapes=[pltpu.VMEM((8, 128), jnp.float32)],
    )(acc1, W1, b1, g1, be1, W2, a_src2, a_dst2)
    carr2 = c2out[:2, :16]

    acc2 = _sc_pass2(t2a, t2b, idx2, carr2, zeros, npad, steps2)

    y = pl.pallas_call(
        _final_body,
        grid=(npad // rblk,),
        in_specs=[
            pl.BlockSpec((2, rblk, 32), lambda i: (0, i, 0)),
            pl.BlockSpec((16,), lambda i: (0,)),
            pl.BlockSpec((16, 1), lambda i: (0, 0)),
            pl.BlockSpec((1,), lambda i: (0,)),
        ],
        out_specs=pl.BlockSpec((rblk, 1), lambda i: (i, 0)),
        out_shape=jax.ShapeDtypeStruct((npad, 1), jnp.float32),
    )(acc2, b2, Wc, bc)
    return y[:n]


# speedup vs baseline: 1.0379x; 1.0379x over previous
import dataclasses
import functools

import jax
import jax.numpy as jnp
from jax import lax
from jax.experimental import pallas as pl
from jax.experimental.pallas import tpu as pltpu
from jax.experimental.pallas import tpu_sc as plsc

_NC = 2
_NS = 16
_NW = _NC * _NS
_LANES = 16
_CHUNK1 = 256
_SUB1 = _CHUNK1 // 128
_CHUNK2 = 128


def _sc_compiler_params():
    cp = pltpu.CompilerParams()
    fields = pltpu.CompilerParams.__dataclass_fields__
    if "needs_layout_passes" in fields:
        cp = dataclasses.replace(cp, needs_layout_passes=False)
    if "use_tc_tiling_on_sc" in fields:
        cp = dataclasses.replace(cp, use_tc_tiling_on_sc=False)
    return cp


def _dot(a, b):
    return jnp.dot(a, b, precision=jax.lax.Precision.HIGHEST,
                   preferred_element_type=jnp.float32)


def _elu(v):
    return jnp.where(v > 0, v, jnp.exp(jnp.minimum(v, 0.0)) - 1.0)


def _lrelu(v):
    return jnp.maximum(v, 0.2 * v)


def _cmax_update(i, cacc_ref, bmv):
    @pl.when(i == 0)
    def _():
        cacc_ref[...] = bmv

    @pl.when(i > 0)
    def _():
        cacc_ref[...] = jnp.maximum(cacc_ref[...], bmv)


def _prep1_body(x_ref, w1_ref, as1_ref, ad1_ref, t1_ref, c_ref, cacc_ref):
    i = pl.program_id(0)
    x = x_ref[...]
    w1 = w1_ref[...]
    vs = jnp.stack([_dot(w1[:, :32], as1_ref[0, :]),
                    _dot(w1[:, 32:], as1_ref[1, :])], axis=1)
    vd = jnp.stack([_dot(w1[:, :32], ad1_ref[0, :]),
                    _dot(w1[:, 32:], ad1_ref[1, :])], axis=1)
    es = _dot(x, vs)
    ed = _dot(x, vd)
    rblk = x.shape[0]
    one = jnp.ones((rblk, 1), jnp.float32)
    zero = jnp.zeros((rblk, 1), jnp.float32)
    t1_ref[...] = jnp.concatenate([x, one, es, ed, zero], axis=1)
    bm = jnp.stack([jnp.max(es[:, 0]), jnp.max(es[:, 1]),
                    jnp.max(ed[:, 0]), jnp.max(ed[:, 1])])
    bm8 = jnp.concatenate([bm, jnp.zeros((4,), jnp.float32)])
    _cmax_update(i, cacc_ref, jnp.broadcast_to(bm8[:, None], (8, 128)))

    @pl.when(i == pl.num_programs(0) - 1)
    def _():
        r0 = _lrelu(cacc_ref[0, :] + cacc_ref[2, :])
        r1 = _lrelu(cacc_ref[1, :] + cacc_ref[3, :])
        z = jnp.zeros((128,), jnp.float32)
        c_ref[...] = jnp.stack([r0, r1, z, z, z, z, z, z])


def _prep2_body(n, rblk, acc_ref, w1_ref, b1_ref, g1_ref, be1_ref, w2_ref,
                as2_ref, ad2_ref, t2a_ref, t2b_ref, c_ref, cacc_ref):
    s = acc_ref[0] + acc_ref[1]
    m1 = s[:, 0:10]
    d1 = s[:, 10:11]
    m2 = s[:, 16:26]
    d2 = s[:, 26:27]
    w1 = w1_ref[...]
    o1 = _dot(m1, w1[:, :32])
    o2 = _dot(m2, w1[:, 32:])
    o1 = jnp.where(d1 > 0, o1 / d1, 0.0)
    o2 = jnp.where(d2 > 0, o2 / d2, 0.0)
    o = jnp.concatenate([o1, o2], axis=1) + b1_ref[...][None, :]
    z = o * (g1_ref[...][None, :] / jnp.sqrt(1.0 + 1e-5)) + be1_ref[...][None, :]
    z = _elu(z)
    h2 = _dot(z, w2_ref[...])
    es2 = _dot(h2, as2_ref[0, :])
    ed2 = _dot(h2, ad2_ref[0, :])
    i = pl.program_id(0)
    row = lax.broadcasted_iota(jnp.int32, (rblk, 1), 0) + i * rblk
    mask = row < n
    h2 = jnp.where(mask, h2, 0.0)
    es2 = jnp.where(mask[:, 0], es2, 0.0)
    ed2 = jnp.where(mask[:, 0], ed2, 0.0)
    zc = jnp.zeros((rblk, 1), jnp.float32)
    one = jnp.where(mask, 1.0, 0.0)
    t2a_ref[...] = jnp.concatenate(
        [h2, es2[:, None], jnp.zeros((rblk, 15), jnp.float32)], axis=1)
    t2b_ref[...] = jnp.concatenate(
        [zc, ed2[:, None], one, jnp.zeros((rblk, 13), jnp.float32)], axis=1)
    bm = jnp.stack([jnp.max(es2), jnp.max(ed2)])
    bm8 = jnp.concatenate([bm, jnp.zeros((6,), jnp.float32)])
    _cmax_update(i, cacc_ref, jnp.broadcast_to(bm8[:, None], (8, 128)))

    @pl.when(i == pl.num_programs(0) - 1)
    def _():
        r0 = _lrelu(cacc_ref[0, :] + cacc_ref[1, :])
        z = jnp.zeros((128,), jnp.float32)
        c_ref[...] = jnp.stack([r0, z, z, z, z, z, z, z])


def _final_body(acc_ref, b2_ref, wc_ref, bc_ref, y_ref):
    s = acc_ref[0] + acc_ref[1]
    m = s[:, 0:16]
    d = s[:, 18:19]
    o = jnp.where(d > 0, m / d, 0.0) + b2_ref[...][None, :]
    o = _elu(o)
    y_ref[...] = _dot(o, wc_ref[...]) + bc_ref[...][None, :]


def _sc_pass1(t1, idx4, carr, zeros, npad, steps):
    rpt = npad // _NS
    mesh = plsc.VectorSubcoreMesh(core_axis_name="c", subcore_axis_name="s")

    @functools.partial(
        pl.kernel,
        out_type=jax.ShapeDtypeStruct((_NC, npad, 32), jnp.float32),
        mesh=mesh,
        compiler_params=_sc_compiler_params(),
        scratch_types=[
            pltpu.VMEM((4, 128), jnp.int32),
            pltpu.VMEM((4, 128), jnp.int32),
            pltpu.VMEM((_CHUNK1, 16), jnp.float32),
            pltpu.VMEM((_CHUNK1, 16), jnp.float32),
            pltpu.VMEM((_CHUNK1, 16), jnp.float32),
            pltpu.VMEM((_CHUNK1, 16), jnp.float32),
            pltpu.VMEM((_CHUNK1, 32), jnp.float32),
            pltpu.VMEM((2, 16), jnp.float32),
            pltpu.VMEM_SHARED((npad, 32), jnp.float32),
            pltpu.SemaphoreType.DMA,
            pltpu.SemaphoreType.DMA,
        ],
    )
    def kern(t1_hbm, idx_hbm, c_hbm, z_hbm, out_hbm,
             sidi0, sidi1, srows0, srows1, drows0, drows1, msg,
             cbuf, acc, g0, g1):
        cid = lax.axis_index("c")
        sid = lax.axis_index("s")
        w = cid * _NS + sid
        pltpu.sync_copy(z_hbm.at[pl.ds(sid * rpt, rpt)],
                        acc.at[pl.ds(sid * rpt, rpt)])
        pltpu.sync_copy(c_hbm, cbuf)
        plsc.subcore_barrier()
        iota = lax.iota(jnp.int32, _LANES)
        c1v = cbuf[0, :]
        c2v = cbuf[1, :]
        col11 = jnp.full((_LANES,), 11, jnp.int32)
        col12 = jnp.full((_LANES,), 12, jnp.int32)
        col13 = jnp.full((_LANES,), 13, jnp.int32)
        col14 = jnp.full((_LANES,), 14, jnp.int32)
        bufs = [(sidi0, srows0, drows0, g0),
                (sidi1, srows1, drows1, g1)]

        def prefetch(step, b):
            sidi, srows, drows, sem = bufs[b]
            base = (w * steps + step) * 4
            pltpu.sync_copy(idx_hbm.at[pl.ds(base, 4)], sidi)
            cps = []
            for j in range(_SUB1):
                cps.append(pltpu.async_copy(t1_hbm.at[sidi.at[j]],
                                            srows.at[pl.ds(j * 128, 128)],
                                            sem))
                cps.append(pltpu.async_copy(t1_hbm.at[sidi.at[2 + j]],
                                            drows.at[pl.ds(j * 128, 128)],
                                            sem))
            return cps

        def process(b, cps):
            sidi, srows, drows, _ = bufs[b]
            for cp in cps:
                cp.wait()

            @pl.loop(0, _CHUNK1 // _LANES)
            def _(g):
                r0 = g * _LANES
                rows = r0 + iota
                es1 = plsc.load_gather(srows, [rows, col11])
                es2 = plsc.load_gather(srows, [rows, col12])
                ed1 = plsc.load_gather(drows, [rows, col13])
                ed2 = plsc.load_gather(drows, [rows, col14])
                e1 = es1 + ed1
                e1 = jnp.maximum(e1, 0.2 * e1)
                e2 = es2 + ed2
                e2 = jnp.maximum(e2, 0.2 * e2)
                p1 = jnp.exp(e1 - c1v)
                p2 = jnp.exp(e2 - c2v)
                for i in range(_LANES):
                    srow = srows[r0 + i, :]
                    msg[r0 + i, 0:16] = srow * p1[i]
                    msg[r0 + i, 16:32] = srow * p2[i]

            for j in range(_SUB1):
                pltpu.sync_copy(msg.at[pl.ds(j * 128, 128)],
                                acc.at[sidi.at[2 + j]], add=True)

        @pl.loop(0, steps // 2)
        def _(k):
            s0 = 2 * k
            h0 = prefetch(s0, 0)
            h1 = prefetch(s0 + 1, 1)
            process(0, h0)
            process(1, h1)

        plsc.subcore_barrier()
        pltpu.sync_copy(acc.at[pl.ds(sid * rpt, rpt)],
                        out_hbm.at[cid, pl.ds(sid * rpt, rpt)])

    return kern(t1, idx4, carr, zeros)


def _sc_pass2(t2a, t2b, idx2, carr, zeros, npad, steps2):
    rpt = npad // _NS
    mesh = plsc.VectorSubcoreMesh(core_axis_name="c", subcore_axis_name="s")

    @functools.partial(
        pl.kernel,
        out_type=jax.ShapeDtypeStruct((_NC, npad, 32), jnp.float32),
        mesh=mesh,
        compiler_params=_sc_compiler_params(),
        scratch_types=[
            pltpu.VMEM((2, 128), jnp.int32),
            pltpu.VMEM((2, 128), jnp.int32),
            pltpu.VMEM((_CHUNK2, 32), jnp.float32),
            pltpu.VMEM((_CHUNK2, 32), jnp.float32),
            pltpu.VMEM((_CHUNK2, 16), jnp.float32),
            pltpu.VMEM((_CHUNK2, 16), jnp.float32),
            pltpu.VMEM((_CHUNK2, 32), jnp.float32),
            pltpu.VMEM((2, 16), jnp.float32),
            pltpu.VMEM_SHARED((npad, 32), jnp.float32),
            pltpu.SemaphoreType.DMA,
            pltpu.SemaphoreType.DMA,
        ],
    )
    def kern(t2a_hbm, t2b_hbm, idx_hbm, c_hbm, z_hbm, out_hbm,
             sidi0, sidi1, ar0, ar1, cr0, cr1, msg,
             cbuf, acc, g0, g1):
        cid = lax.axis_index("c")
        sid = lax.axis_index("s")
        w = cid * _NS + sid
        pltpu.sync_copy(z_hbm.at[pl.ds(sid * rpt, rpt)],
                        acc.at[pl.ds(sid * rpt, rpt)])
        pltpu.sync_copy(c_hbm, cbuf)
        plsc.subcore_barrier()
        iota = lax.iota(jnp.int32, _LANES)
        cv = cbuf[0, :]
        col16 = jnp.full((_LANES,), 16, jnp.int32)
        col1 = jnp.full((_LANES,), 1, jnp.int32)
        bufs = [(sidi0, ar0, cr0, g0),
                (sidi1, ar1, cr1, g1)]

        def prefetch(step, b):
            sidi, ar, cr, sem = bufs[b]
            base = (w * steps2 + step) * 2
            pltpu.sync_copy(idx_hbm.at[pl.ds(base, 2)], sidi)
            return [
                pltpu.async_copy(t2a_hbm.at[sidi.at[0]], ar, sem),
                pltpu.async_copy(t2b_hbm.at[sidi.at[1]], cr, sem),
            ]

        def process(b, cps):
            sidi, ar, cr, _ = bufs[b]
            for cp in cps:
                cp.wait()

            @pl.loop(0, _CHUNK2 // _LANES)
            def _(g):
                r0 = g * _LANES
                rows = r0 + iota
                es = plsc.load_gather(ar, [rows, col16])
                ed = plsc.load_gather(cr, [rows, col1])
                e = es + ed
                e = jnp.maximum(e, 0.2 * e)
                pv = jnp.exp(e - cv)
                for i in range(_LANES):
                    p = pv[i]
                    msg[r0 + i, 0:16] = ar[r0 + i, 0:16] * p
                    msg[r0 + i, 16:32] = cr[r0 + i, :] * p

            pltpu.sync_copy(msg, acc.at[sidi.at[1]], add=True)

        @pl.loop(0, steps2 // 2)
        def _(k):
            s0 = 2 * k
            h0 = prefetch(s0, 0)
            h1 = prefetch(s0 + 1, 1)
            process(0, h0)
            process(1, h1)

        plsc.subcore_barrier()
        pltpu.sync_copy(acc.at[pl.ds(sid * rpt, rpt)],
                        out_hbm.at[cid, pl.ds(sid * rpt, rpt)])

    return kern(t2a, t2b, idx2, carr, zeros)


def kernel(x, edge_index, W1, a_src1, a_dst1, b1, g1, be1,
           W2, a_src2, a_dst2, b2, Wc, bc):
    n = x.shape[0]
    e = edge_index.shape[1]
    npad = ((n + 1 + 127) // 128) * 128
    two = 2 * _CHUNK1
    ew = ((e + _NW * two - 1) // (_NW * two)) * two
    steps = ew // _CHUNK1
    steps2 = ew // _CHUNK2
    ep = ew * _NW

    src = edge_index[0]
    dst = edge_index[1]
    pad = jnp.full((ep - e,), n, jnp.int32)
    srcp = jnp.concatenate([src, pad])
    dstp = jnp.concatenate([dst, pad])
    srcr = srcp.reshape(_NW * steps, _SUB1, 128)
    dstr = dstp.reshape(_NW * steps, _SUB1, 128)
    idx4 = jnp.concatenate([srcr, dstr], axis=1).reshape(_NW * steps * 4, 128)
    srcr2 = srcp.reshape(_NW * steps2, 1, 128)
    dstr2 = dstp.reshape(_NW * steps2, 1, 128)
    idx2 = jnp.concatenate([srcr2, dstr2], axis=1).reshape(_NW * steps2 * 2,
                                                           128)
    xp = jnp.pad(x, ((0, npad - n), (0, 0)))
    zeros = jnp.zeros((npad, 32), jnp.float32)

    rblk = npad // 8
    t1, c1out = pl.pallas_call(
        _prep1_body,
        grid=(npad // rblk,),
        in_specs=[
            pl.BlockSpec((rblk, 10), lambda i: (i, 0)),
            pl.BlockSpec((10, 64), lambda i: (0, 0)),
            pl.BlockSpec((2, 32), lambda i: (0, 0)),
            pl.BlockSpec((2, 32), lambda i: (0, 0)),
        ],
        out_specs=[
            pl.BlockSpec((rblk, 16), lambda i: (i, 0)),
            pl.BlockSpec((8, 128), lambda i: (0, 0)),
        ],
        out_shape=[
            jax.ShapeDtypeStruct((npad, 16), jnp.float32),
            jax.ShapeDtypeStruct((8, 128), jnp.float32),
        ],
        scratch_shapes=[pltpu.VMEM((8, 128), jnp.float32)],
    )(xp, W1, a_src1, a_dst1)
    carr1 = c1out[:2, :16]

    acc1 = _sc_pass1(t1, idx4, carr1, zeros, npad, steps)

    t2a, t2b, c2out = pl.pallas_call(
        functools.partial(_prep2_body, n, rblk),
        grid=(npad // rblk,),
        in_specs=[
            pl.BlockSpec((2, rblk, 32), lambda i: (0, i, 0)),
            pl.BlockSpec((10, 64), lambda i: (0, 0)),
            pl.BlockSpec((64,), lambda i: (0,)),
            pl.BlockSpec((64,), lambda i: (0,)),
            pl.BlockSpec((64,), lambda i: (0,)),
            pl.BlockSpec((64, 16), lambda i: (0, 0)),
            pl.BlockSpec((1, 16), lambda i: (0, 0)),
            pl.BlockSpec((1, 16), lambda i: (0, 0)),
        ],
        out_specs=[
            pl.BlockSpec((rblk, 32), lambda i: (i, 0)),
            pl.BlockSpec((rblk, 16), lambda i: (i, 0)),
            pl.BlockSpec((8, 128), lambda i: (0, 0)),
        ],
        out_shape=[
            jax.ShapeDtypeStruct((npad, 32), jnp.float32),
            jax.ShapeDtypeStruct((npad, 16), jnp.float32),
            jax.ShapeDtypeStruct((8, 128), jnp.float32),
        ],
        scratch_shapes=[pltpu.VMEM((8, 128), jnp.float32)],
    )(acc1, W1, b1, g1, be1, W2, a_src2, a_dst2)
    carr2 = c2out[:2, :16]

    acc2 = _sc_pass2(t2a, t2b, idx2, carr2, zeros, npad, steps2)

    y = pl.pallas_call(
        _final_body,
        grid=(npad // rblk,),
        in_specs=[
            pl.BlockSpec((2, rblk, 32), lambda i: (0, i, 0)),
            pl.BlockSpec((16,), lambda i: (0,)),
            pl.BlockSpec((16, 1), lambda i: (0, 0)),
            pl.BlockSpec((1,), lambda i: (0,)),
        ],
        out_specs=pl.BlockSpec((rblk, 1), lambda i: (i, 0)),
        out_shape=jax.ShapeDtypeStruct((npad, 1), jnp.float32),
    )(acc2, b2, Wc, bc)
    return y[:n]
